# baseline (speedup 1.0000x reference)
import functools
import jax
import jax.numpy as jnp
from jax import lax
from jax.experimental import pallas as pl
from jax.experimental.pallas import tpu as pltpu
from jax.experimental.pallas import tpu_sc as plsc

N_NODES = 10000
N_EDGES = 320000
D = 128
NC, NS = 2, 16
NW = NC * NS
CH = 80
NM = 16
KI = 8
KR = 4
EPTP = NM * KI * CH
PAD = NW * EPTP - N_EDGES
NCHP = NM * KI
DUMP = N_NODES
ACC_R = N_NODES + 8
KBD = 10
NBD = NCHP // KBD
KTD = NCHP - NBD * KBD
BLK = 40
NBLK = N_NODES // BLK
BPT = -(-NBLK // NS)

_mesh = plsc.VectorSubcoreMesh(core_axis_name="c", subcore_axis_name="s")


def _zero_vmem_2d(ref, nrows, ncols):
    z16 = jnp.zeros((16,), jnp.float32)

    def row(r, carry):
        for j in range(ncols // 16):
            ref[r, pl.ds(j * 16, 16)] = z16
        return carry

    lax.fori_loop(0, nrows, row, 0)


def _blocks(s, fn):
    def body(j, carry):
        bi = s + j * NS

        @pl.when(bi < NBLK)
        def _():
            fn(pl.multiple_of(bi * BLK, 8))

        return carry

    lax.fori_loop(0, BPT, body, 0)


@functools.partial(
    pl.kernel,
    out_type=jax.ShapeDtypeStruct((NC, N_NODES, D), jnp.float32),
    mesh=_mesh,
    scratch_types=[
        pltpu.VMEM((NCHP, CH), jnp.int32),
        pltpu.VMEM((CH, D), jnp.float32),
        pltpu.VMEM((BLK, D), jnp.float32),
        pltpu.VMEM_SHARED((ACC_R, D), jnp.float32),
        pltpu.SemaphoreType.DMA,
    ],
)
def _sc_degree(dst_hbm, out_hbm, idx_v, ones_v, zero_v, acc_sh, sem):
    c = lax.axis_index("c")
    s = lax.axis_index("s")
    wid = c * NS + s

    one16 = jnp.ones((16,), jnp.float32)

    def fill_ones(r, carry):
        for j in range(D // 16):
            ones_v[r, pl.ds(j * 16, 16)] = one16
        return carry

    lax.fori_loop(0, CH, fill_ones, 0)
    _zero_vmem_2d(zero_v, BLK, D)
    _blocks(s, lambda off: pltpu.sync_copy(zero_v, acc_sh.at[pl.ds(off, BLK)]))
    pltpu.sync_copy(dst_hbm.at[wid], idx_v)
    plsc.subcore_barrier()

    def batch(i, nch):
        ds = [pltpu.async_copy(ones_v, acc_sh.at[idx_v.at[i * KBD + b]],
                               sem, add=True) for b in range(nch)]
        for d in ds:
            d.wait()

    def body(i, carry):
        batch(i, KBD)
        return carry

    lax.fori_loop(0, NBD, body, 0)
    if KTD:
        batch(NBD, KTD)
    plsc.subcore_barrier()
    _blocks(s, lambda off: pltpu.sync_copy(
        acc_sh.at[pl.ds(off, BLK)], out_hbm.at[c, pl.ds(off, BLK)]))


@functools.partial(
    pl.kernel,
    out_type=jax.ShapeDtypeStruct((NC, N_NODES, D), jnp.float32),
    mesh=_mesh,
    scratch_types=[
        pltpu.VMEM((KI, CH), jnp.int32),
        pltpu.VMEM((KI, CH), jnp.int32),
        pltpu.VMEM((KR, CH, D), jnp.float32),
        pltpu.VMEM_SHARED((ACC_R, D), jnp.float32),
        pltpu.SemaphoreType.DMA,
        pltpu.SemaphoreType.DMA,
    ],
)
def _sc_scatter(g_hbm, src_hbm, dst_hbm, out_hbm,
                isrc_v, idst_v, rows_v, acc_sh, gsem, ssem):
    c = lax.axis_index("c")
    s = lax.axis_index("s")
    wid = c * NS + s

    _zero_vmem_2d(rows_v.at[0], BLK, D)
    _blocks(s, lambda off: pltpu.sync_copy(rows_v.at[0, pl.ds(0, BLK)],
                                           acc_sh.at[pl.ds(off, BLK)]))
    plsc.subcore_barrier()

    def body(m, carry):
        pltpu.sync_copy(src_hbm.at[wid, m], isrc_v)
        pltpu.sync_copy(dst_hbm.at[wid, m], idst_v)
        for h in range(KI // KR):
            gds = [pltpu.async_copy(g_hbm.at[isrc_v.at[h * KR + b]],
                                    rows_v.at[b], gsem) for b in range(KR)]
            for d in gds:
                d.wait()
            sds = [pltpu.async_copy(rows_v.at[b],
                                    acc_sh.at[idst_v.at[h * KR + b]],
                                    ssem, add=True) for b in range(KR)]
            for d in sds:
                d.wait()
        return carry

    lax.fori_loop(0, NM, body, 0)
    plsc.subcore_barrier()
    _blocks(s, lambda off: pltpu.sync_copy(
        acc_sh.at[pl.ds(off, BLK)], out_hbm.at[c, pl.ds(off, BLK)]))


_BR = 1000
_GRID = N_NODES // _BR


def _dis_of(degp):
    return lax.rsqrt(degp[0] + degp[1] + 1.0)


def _tc1_body(x_ref, w_ref, degp_ref, o_ref):
    dis = _dis_of(degp_ref[...])
    h = jnp.dot(x_ref[...], w_ref[...], preferred_element_type=jnp.float32)
    o_ref[...] = h * dis


def _tc2_body(s_ref, g_ref, degp_ref, b_ref, w_ref, o_ref):
    dis = _dis_of(degp_ref[...])
    tot = s_ref[0] + s_ref[1] + g_ref[...]
    z = jnp.maximum(tot * dis + b_ref[...], 0.0)
    h = jnp.dot(z, w_ref[...], preferred_element_type=jnp.float32)
    o_ref[...] = h * dis


def _tc3_body(s_ref, g_ref, degp_ref, b_ref, o_ref):
    dis = _dis_of(degp_ref[...])
    tot = s_ref[0] + s_ref[1] + g_ref[...]
    o_ref[...] = jnp.maximum(tot * dis + b_ref[...], 0.0)


_spec_rows = pl.BlockSpec((_BR, D), lambda i: (i, 0))
_spec_S = pl.BlockSpec((NC, _BR, D), lambda i: (0, i, 0))
_spec_degp = pl.BlockSpec((NC, _BR, D), lambda i: (0, i, 0))
_spec_w = pl.BlockSpec((D, D), lambda i: (0, 0))
_spec_b = pl.BlockSpec((1, D), lambda i: (0, 0))

_tc1 = pl.pallas_call(
    _tc1_body, grid=(_GRID,),
    in_specs=[_spec_rows, _spec_w, _spec_degp],
    out_specs=_spec_rows,
    out_shape=jax.ShapeDtypeStruct((N_NODES, D), jnp.float32),
)
_tc2 = pl.pallas_call(
    _tc2_body, grid=(_GRID,),
    in_specs=[_spec_S, _spec_rows, _spec_degp, _spec_b, _spec_w],
    out_specs=_spec_rows,
    out_shape=jax.ShapeDtypeStruct((N_NODES, D), jnp.float32),
)
_tc3 = pl.pallas_call(
    _tc3_body, grid=(_GRID,),
    in_specs=[_spec_S, _spec_rows, _spec_degp, _spec_b],
    out_specs=_spec_rows,
    out_shape=jax.ShapeDtypeStruct((N_NODES, D), jnp.float32),
)


@jax.jit
def kernel(x, edge_index, W1, b1, W2, b2):
    src = edge_index[0].astype(jnp.int32)
    dst = edge_index[1].astype(jnp.int32)
    src_p = jnp.concatenate([src, jnp.zeros((PAD,), jnp.int32)])
    dst_p = jnp.concatenate([dst, jnp.full((PAD,), DUMP, jnp.int32)])
    src4 = src_p.reshape(NW, NM, KI, CH)
    dst4 = dst_p.reshape(NW, NM, KI, CH)
    dst3 = dst_p.reshape(NW, NCHP, CH)
    b1r = b1.reshape(1, D)
    b2r = b2.reshape(1, D)

    degp = _sc_degree(dst3)
    g1 = _tc1(x, W1, degp)
    S1 = _sc_scatter(g1, src4, dst4)
    g2 = _tc2(S1, g1, degp, b1r, W2)
    S2 = _sc_scatter(g2, src4, dst4)
    return _tc3(S2, g2, degp, b2r)


# speedup vs baseline: 1.0179x; 1.0179x over previous
import functools
import jax
import jax.numpy as jnp
from jax import lax
from jax.experimental import pallas as pl
from jax.experimental.pallas import tpu as pltpu
from jax.experimental.pallas import tpu_sc as plsc

N_NODES = 10000
N_EDGES = 320000
D = 128
NC, NS = 2, 16
NW = NC * NS
CH = 80
NM = 16
KI = 8
KR = 4
EPTP = NM * KI * CH
PAD = NW * EPTP - N_EDGES
NCHP = NM * KI
NDUMP = 512
ACC_R = N_NODES + NDUMP
KBD = 10
NBD = NCHP // KBD
KTD = NCHP - NBD * KBD
BLK = 40
NBLK = N_NODES // BLK
BPT = -(-NBLK // NS)

_mesh = plsc.VectorSubcoreMesh(core_axis_name="c", subcore_axis_name="s")


def _zero_vmem_2d(ref, nrows, ncols):
    z16 = jnp.zeros((16,), jnp.float32)

    def row(r, carry):
        for j in range(ncols // 16):
            ref[r, pl.ds(j * 16, 16)] = z16
        return carry

    lax.fori_loop(0, nrows, row, 0)


def _blocks(s, fn):
    def body(j, carry):
        bi = s + j * NS

        @pl.when(bi < NBLK)
        def _():
            fn(pl.multiple_of(bi * BLK, 8))

        return carry

    lax.fori_loop(0, BPT, body, 0)


@functools.partial(
    pl.kernel,
    out_type=jax.ShapeDtypeStruct((NC, N_NODES, D), jnp.float32),
    mesh=_mesh,
    scratch_types=[
        pltpu.VMEM((NCHP, CH), jnp.int32),
        pltpu.VMEM((CH, D), jnp.float32),
        pltpu.VMEM((BLK, D), jnp.float32),
        pltpu.VMEM_SHARED((ACC_R, D), jnp.float32),
        pltpu.SemaphoreType.DMA,
    ],
)
def _sc_degree(dst_hbm, out_hbm, idx_v, ones_v, zero_v, acc_sh, sem):
    c = lax.axis_index("c")
    s = lax.axis_index("s")
    wid = c * NS + s

    one16 = jnp.ones((16,), jnp.float32)

    def fill_ones(r, carry):
        for j in range(D // 16):
            ones_v[r, pl.ds(j * 16, 16)] = one16
        return carry

    lax.fori_loop(0, CH, fill_ones, 0)
    _zero_vmem_2d(zero_v, BLK, D)
    _blocks(s, lambda off: pltpu.sync_copy(zero_v, acc_sh.at[pl.ds(off, BLK)]))
    pltpu.sync_copy(dst_hbm.at[wid], idx_v)
    plsc.subcore_barrier()

    def batch(i, nch):
        ds = [pltpu.async_copy(ones_v, acc_sh.at[idx_v.at[i * KBD + b]],
                               sem, add=True) for b in range(nch)]
        for d in ds:
            d.wait()

    def body(i, carry):
        batch(i, KBD)
        return carry

    lax.fori_loop(0, NBD, body, 0)
    if KTD:
        batch(NBD, KTD)
    plsc.subcore_barrier()
    _blocks(s, lambda off: pltpu.sync_copy(
        acc_sh.at[pl.ds(off, BLK)], out_hbm.at[c, pl.ds(off, BLK)]))


@functools.partial(
    pl.kernel,
    out_type=jax.ShapeDtypeStruct((NC, N_NODES, D), jnp.float32),
    mesh=_mesh,
    scratch_types=[
        pltpu.VMEM((KI, CH), jnp.int32),
        pltpu.VMEM((KI, CH), jnp.int32),
        pltpu.VMEM((KR, CH, D), jnp.float32),
        pltpu.VMEM_SHARED((ACC_R, D), jnp.float32),
        pltpu.SemaphoreType.DMA,
        pltpu.SemaphoreType.DMA,
    ],
)
def _sc_scatter(g_hbm, src_hbm, dst_hbm, out_hbm,
                isrc_v, idst_v, rows_v, acc_sh, gsem, ssem):
    c = lax.axis_index("c")
    s = lax.axis_index("s")
    wid = c * NS + s

    _zero_vmem_2d(rows_v.at[0], BLK, D)
    _blocks(s, lambda off: pltpu.sync_copy(rows_v.at[0, pl.ds(0, BLK)],
                                           acc_sh.at[pl.ds(off, BLK)]))
    plsc.subcore_barrier()

    def body(m, carry):
        pltpu.sync_copy(src_hbm.at[wid, m], isrc_v)
        pltpu.sync_copy(dst_hbm.at[wid, m], idst_v)

        def gath(ch, buf):
            return [pltpu.async_copy(g_hbm.at[isrc_v.at[ch + i]],
                                     rows_v.at[buf + i], gsem)
                    for i in range(2)]

        def scat(ch, buf):
            return [pltpu.async_copy(rows_v.at[buf + i],
                                     acc_sh.at[idst_v.at[ch + i]],
                                     ssem, add=True) for i in range(2)]

        g01 = gath(0, 0)
        for d in g01:
            d.wait()
        s01 = scat(0, 0)
        g23 = gath(2, 2)
        for d in g23:
            d.wait()
        s23 = scat(2, 2)
        for d in s01:
            d.wait()
        g45 = gath(4, 0)
        for d in g45:
            d.wait()
        s45 = scat(4, 0)
        for d in s23:
            d.wait()
        g67 = gath(6, 2)
        for d in g67:
            d.wait()
        s67 = scat(6, 2)
        for d in s45:
            d.wait()
        for d in s67:
            d.wait()
        return carry

    lax.fori_loop(0, NM, body, 0)
    plsc.subcore_barrier()
    _blocks(s, lambda off: pltpu.sync_copy(
        acc_sh.at[pl.ds(off, BLK)], out_hbm.at[c, pl.ds(off, BLK)]))


_BR = 1000
_GRID = N_NODES // _BR


def _dis_of(degp):
    return lax.rsqrt(degp[0] + degp[1] + 1.0)


def _tc1_body(x_ref, w_ref, degp_ref, o_ref):
    dis = _dis_of(degp_ref[...])
    h = jnp.dot(x_ref[...], w_ref[...], preferred_element_type=jnp.float32)
    o_ref[...] = h * dis


def _tc2_body(s_ref, g_ref, degp_ref, b_ref, w_ref, o_ref):
    dis = _dis_of(degp_ref[...])
    tot = s_ref[0] + s_ref[1] + g_ref[...]
    z = jnp.maximum(tot * dis + b_ref[...], 0.0)
    h = jnp.dot(z, w_ref[...], preferred_element_type=jnp.float32)
    o_ref[...] = h * dis


def _tc3_body(s_ref, g_ref, degp_ref, b_ref, o_ref):
    dis = _dis_of(degp_ref[...])
    tot = s_ref[0] + s_ref[1] + g_ref[...]
    o_ref[...] = jnp.maximum(tot * dis + b_ref[...], 0.0)


_spec_rows = pl.BlockSpec((_BR, D), lambda i: (i, 0))
_spec_S = pl.BlockSpec((NC, _BR, D), lambda i: (0, i, 0))
_spec_degp = pl.BlockSpec((NC, _BR, D), lambda i: (0, i, 0))
_spec_w = pl.BlockSpec((D, D), lambda i: (0, 0))
_spec_b = pl.BlockSpec((1, D), lambda i: (0, 0))

_tc1 = pl.pallas_call(
    _tc1_body, grid=(_GRID,),
    in_specs=[_spec_rows, _spec_w, _spec_degp],
    out_specs=_spec_rows,
    out_shape=jax.ShapeDtypeStruct((N_NODES, D), jnp.float32),
)
_tc2 = pl.pallas_call(
    _tc2_body, grid=(_GRID,),
    in_specs=[_spec_S, _spec_rows, _spec_degp, _spec_b, _spec_w],
    out_specs=_spec_rows,
    out_shape=jax.ShapeDtypeStruct((N_NODES, D), jnp.float32),
)
_tc3 = pl.pallas_call(
    _tc3_body, grid=(_GRID,),
    in_specs=[_spec_S, _spec_rows, _spec_degp, _spec_b],
    out_specs=_spec_rows,
    out_shape=jax.ShapeDtypeStruct((N_NODES, D), jnp.float32),
)


@jax.jit
def kernel(x, edge_index, W1, b1, W2, b2):
    src = edge_index[0].astype(jnp.int32)
    dst = edge_index[1].astype(jnp.int32)
    src_p = jnp.concatenate([src, jnp.zeros((PAD,), jnp.int32)])
    dump = N_NODES + jnp.arange(PAD, dtype=jnp.int32) % NDUMP
    dst_p = jnp.concatenate([dst, dump])
    src4 = src_p.reshape(NW, NM, KI, CH)
    dst4 = dst_p.reshape(NW, NM, KI, CH)
    dst3 = dst_p.reshape(NW, NCHP, CH)
    b1r = b1.reshape(1, D)
    b2r = b2.reshape(1, D)

    degp = _sc_degree(dst3)
    g1 = _tc1(x, W1, degp)
    S1 = _sc_scatter(g1, src4, dst4)
    g2 = _tc2(S1, g1, degp, b1r, W2)
    S2 = _sc_scatter(g2, src4, dst4)
    return _tc3(S2, g2, degp, b2r)


# speedup vs baseline: 2.7824x; 2.7333x over previous
import functools
import jax
import jax.numpy as jnp
from jax import lax
from jax.experimental import pallas as pl
from jax.experimental.pallas import tpu as pltpu
from jax.experimental.pallas import tpu_sc as plsc

N_NODES = 10000
N_EDGES = 320000
D = 128
NC, NS = 2, 16
NW = NC * NS
CH = 80
EPT = N_EDGES // NW
NCH = EPT // CH
KI = 8
NM = -(-NCH // KI)
KR = 4
NMF = NCH // KI
KTL = NCH - NMF * KI
EPTP = NM * KI * CH
NCHP = NM * KI
ACC_R = N_NODES
KBD = 10
NBD = NCH // KBD
KTD = NCH - NBD * KBD
BLK = 40
NBLK = N_NODES // BLK
BPT = -(-NBLK // NS)

_mesh = plsc.VectorSubcoreMesh(core_axis_name="c", subcore_axis_name="s")


def _zero_vmem_2d(ref, nrows, ncols):
    z16 = jnp.zeros((16,), jnp.float32)

    def row(r, carry):
        for j in range(ncols // 16):
            ref[r, pl.ds(j * 16, 16)] = z16
        return carry

    lax.fori_loop(0, nrows, row, 0)


def _blocks(s, fn):
    def body(j, carry):
        bi = s + j * NS

        @pl.when(bi < NBLK)
        def _():
            fn(pl.multiple_of(bi * BLK, 8))

        return carry

    lax.fori_loop(0, BPT, body, 0)


@functools.partial(
    pl.kernel,
    out_type=jax.ShapeDtypeStruct((NC, N_NODES, D), jnp.float32),
    mesh=_mesh,
    scratch_types=[
        pltpu.VMEM((NCHP, CH), jnp.int32),
        pltpu.VMEM((CH, D), jnp.float32),
        pltpu.VMEM((BLK, D), jnp.float32),
        pltpu.VMEM_SHARED((ACC_R, D), jnp.float32),
        pltpu.SemaphoreType.DMA,
    ],
)
def _sc_degree(dst_hbm, out_hbm, idx_v, ones_v, zero_v, acc_sh, sem):
    c = lax.axis_index("c")
    s = lax.axis_index("s")
    wid = c * NS + s

    one16 = jnp.ones((16,), jnp.float32)

    def fill_ones(r, carry):
        for j in range(D // 16):
            ones_v[r, pl.ds(j * 16, 16)] = one16
        return carry

    lax.fori_loop(0, CH, fill_ones, 0)
    _zero_vmem_2d(zero_v, BLK, D)
    _blocks(s, lambda off: pltpu.sync_copy(zero_v, acc_sh.at[pl.ds(off, BLK)]))
    pltpu.sync_copy(dst_hbm.at[wid], idx_v)
    plsc.subcore_barrier()

    def batch(i, nch):
        ds = [pltpu.async_copy(ones_v, acc_sh.at[idx_v.at[i * KBD + b]],
                               sem, add=True) for b in range(nch)]
        for d in ds:
            d.wait()

    def body(i, carry):
        batch(i, KBD)
        return carry

    lax.fori_loop(0, NBD, body, 0)
    if KTD:
        batch(NBD, KTD)
    plsc.subcore_barrier()
    _blocks(s, lambda off: pltpu.sync_copy(
        acc_sh.at[pl.ds(off, BLK)], out_hbm.at[c, pl.ds(off, BLK)]))


@functools.partial(
    pl.kernel,
    out_type=jax.ShapeDtypeStruct((NC, N_NODES, D), jnp.float32),
    mesh=_mesh,
    scratch_types=[
        pltpu.VMEM((KI, CH), jnp.int32),
        pltpu.VMEM((KI, CH), jnp.int32),
        pltpu.VMEM((KR, CH, D), jnp.float32),
        pltpu.VMEM_SHARED((ACC_R, D), jnp.float32),
        pltpu.SemaphoreType.DMA,
        pltpu.SemaphoreType.DMA,
    ],
)
def _sc_scatter(g_hbm, src_hbm, dst_hbm, out_hbm,
                isrc_v, idst_v, rows_v, acc_sh, gsem, ssem):
    c = lax.axis_index("c")
    s = lax.axis_index("s")
    wid = c * NS + s

    _zero_vmem_2d(rows_v.at[0], BLK, D)
    _blocks(s, lambda off: pltpu.sync_copy(rows_v.at[0, pl.ds(0, BLK)],
                                           acc_sh.at[pl.ds(off, BLK)]))
    plsc.subcore_barrier()

    def body(m, carry):
        pltpu.sync_copy(src_hbm.at[wid, m], isrc_v)
        pltpu.sync_copy(dst_hbm.at[wid, m], idst_v)

        def gath(ch, buf):
            return [pltpu.async_copy(g_hbm.at[isrc_v.at[ch + i]],
                                     rows_v.at[buf + i], gsem)
                    for i in range(2)]

        def scat(ch, buf):
            return [pltpu.async_copy(rows_v.at[buf + i],
                                     acc_sh.at[idst_v.at[ch + i]],
                                     ssem, add=True) for i in range(2)]

        g01 = gath(0, 0)
        for d in g01:
            d.wait()
        s01 = scat(0, 0)
        g23 = gath(2, 2)
        for d in g23:
            d.wait()
        s23 = scat(2, 2)
        for d in s01:
            d.wait()
        g45 = gath(4, 0)
        for d in g45:
            d.wait()
        s45 = scat(4, 0)
        for d in s23:
            d.wait()
        g67 = gath(6, 2)
        for d in g67:
            d.wait()
        s67 = scat(6, 2)
        for d in s45:
            d.wait()
        for d in s67:
            d.wait()
        return carry

    lax.fori_loop(0, NMF, body, 0)

    pltpu.sync_copy(src_hbm.at[wid, NMF], isrc_v)
    pltpu.sync_copy(dst_hbm.at[wid, NMF], idst_v)

    def gath1(ch, buf):
        return [pltpu.async_copy(g_hbm.at[isrc_v.at[ch + i]],
                                 rows_v.at[buf + i], gsem) for i in range(2)]

    def scat1(ch, buf):
        return [pltpu.async_copy(rows_v.at[buf + i],
                                 acc_sh.at[idst_v.at[ch + i]],
                                 ssem, add=True) for i in range(2)]

    tg01 = gath1(0, 0)
    for d in tg01:
        d.wait()
    ts01 = scat1(0, 0)
    tg23 = gath1(2, 2)
    for d in tg23:
        d.wait()
    ts23 = scat1(2, 2)
    for d in ts01:
        d.wait()
    tg4 = pltpu.async_copy(g_hbm.at[isrc_v.at[4]], rows_v.at[0], gsem)
    tg4.wait()
    ts4 = pltpu.async_copy(rows_v.at[0], acc_sh.at[idst_v.at[4]],
                           ssem, add=True)
    for d in ts23:
        d.wait()
    ts4.wait()
    plsc.subcore_barrier()
    _blocks(s, lambda off: pltpu.sync_copy(
        acc_sh.at[pl.ds(off, BLK)], out_hbm.at[c, pl.ds(off, BLK)]))


_BR = 1000
_GRID = N_NODES // _BR


def _dis_of(degp):
    return lax.rsqrt(degp[0] + degp[1] + 1.0)


def _tc1_body(x_ref, w_ref, degp_ref, o_ref):
    dis = _dis_of(degp_ref[...])
    h = jnp.dot(x_ref[...], w_ref[...], preferred_element_type=jnp.float32)
    o_ref[...] = h * dis


def _tc2_body(s_ref, g_ref, degp_ref, b_ref, w_ref, o_ref):
    dis = _dis_of(degp_ref[...])
    tot = s_ref[0] + s_ref[1] + g_ref[...]
    z = jnp.maximum(tot * dis + b_ref[...], 0.0)
    h = jnp.dot(z, w_ref[...], preferred_element_type=jnp.float32)
    o_ref[...] = h * dis


def _tc3_body(s_ref, g_ref, degp_ref, b_ref, o_ref):
    dis = _dis_of(degp_ref[...])
    tot = s_ref[0] + s_ref[1] + g_ref[...]
    o_ref[...] = jnp.maximum(tot * dis + b_ref[...], 0.0)


_spec_rows = pl.BlockSpec((_BR, D), lambda i: (i, 0))
_spec_S = pl.BlockSpec((NC, _BR, D), lambda i: (0, i, 0))
_spec_degp = pl.BlockSpec((NC, _BR, D), lambda i: (0, i, 0))
_spec_w = pl.BlockSpec((D, D), lambda i: (0, 0))
_spec_b = pl.BlockSpec((1, D), lambda i: (0, 0))

_tc1 = pl.pallas_call(
    _tc1_body, grid=(_GRID,),
    in_specs=[_spec_rows, _spec_w, _spec_degp],
    out_specs=_spec_rows,
    out_shape=jax.ShapeDtypeStruct((N_NODES, D), jnp.float32),
)
_tc2 = pl.pallas_call(
    _tc2_body, grid=(_GRID,),
    in_specs=[_spec_S, _spec_rows, _spec_degp, _spec_b, _spec_w],
    out_specs=_spec_rows,
    out_shape=jax.ShapeDtypeStruct((N_NODES, D), jnp.float32),
)
_tc3 = pl.pallas_call(
    _tc3_body, grid=(_GRID,),
    in_specs=[_spec_S, _spec_rows, _spec_degp, _spec_b],
    out_specs=_spec_rows,
    out_shape=jax.ShapeDtypeStruct((N_NODES, D), jnp.float32),
)


@jax.jit
def kernel(x, edge_index, W1, b1, W2, b2):
    pad = ((0, 0), (0, EPTP - EPT))
    src_p = jnp.pad(edge_index[0].astype(jnp.int32).reshape(NW, EPT), pad)
    dst_p = jnp.pad(edge_index[1].astype(jnp.int32).reshape(NW, EPT), pad)
    src4 = src_p.reshape(NW, NM, KI, CH)
    dst4 = dst_p.reshape(NW, NM, KI, CH)
    dst3 = dst_p.reshape(NW, NCHP, CH)
    b1r = b1.reshape(1, D)
    b2r = b2.reshape(1, D)

    degp = _sc_degree(dst3)
    g1 = _tc1(x, W1, degp)
    S1 = _sc_scatter(g1, src4, dst4)
    g2 = _tc2(S1, g1, degp, b1r, W2)
    S2 = _sc_scatter(g2, src4, dst4)
    return _tc3(S2, g2, degp, b2r)


# speedup vs baseline: 3.0075x; 1.0809x over previous
import functools
import jax
import jax.numpy as jnp
from jax import lax
from jax.experimental import pallas as pl
from jax.experimental.pallas import tpu as pltpu
from jax.experimental.pallas import tpu_sc as plsc

N_NODES = 10000
N_EDGES = 320000
D = 128
NC, NS = 2, 16
NW = NC * NS
CH = 80
EPT = N_EDGES // NW
NCH = EPT // CH
KI = 8
NM = -(-NCH // KI)
KR = 4
NMF = NCH // KI
KTL = NCH - NMF * KI
EPTP = NM * KI * CH
NCHP = NM * KI
ACC_R = N_NODES
KBD = 10
NBD = NCH // KBD
KTD = NCH - NBD * KBD
BLK = 40
NBLK = N_NODES // BLK
BPT = -(-NBLK // NS)

_mesh = plsc.VectorSubcoreMesh(core_axis_name="c", subcore_axis_name="s")


def _zero_vmem_2d(ref, nrows, ncols):
    z16 = jnp.zeros((16,), jnp.float32)

    def row(r, carry):
        for j in range(ncols // 16):
            ref[r, pl.ds(j * 16, 16)] = z16
        return carry

    lax.fori_loop(0, nrows, row, 0)


def _blocks(s, fn):
    def body(j, carry):
        bi = s + j * NS

        @pl.when(bi < NBLK)
        def _():
            fn(pl.multiple_of(bi * BLK, 8))

        return carry

    lax.fori_loop(0, BPT, body, 0)


@functools.partial(
    pl.kernel,
    out_type=jax.ShapeDtypeStruct((NC, N_NODES, D), jnp.float32),
    mesh=_mesh,
    scratch_types=[
        pltpu.VMEM((NCHP, CH), jnp.int32),
        pltpu.VMEM((CH, D), jnp.float32),
        pltpu.VMEM((BLK, D), jnp.float32),
        pltpu.VMEM_SHARED((ACC_R, D), jnp.float32),
        pltpu.SemaphoreType.DMA,
    ],
)
def _sc_degree(dst_hbm, out_hbm, idx_v, ones_v, zero_v, acc_sh, sem):
    c = lax.axis_index("c")
    s = lax.axis_index("s")
    wid = c * NS + s

    one16 = jnp.ones((16,), jnp.float32)

    def fill_ones(r, carry):
        for j in range(D // 16):
            ones_v[r, pl.ds(j * 16, 16)] = one16
        return carry

    lax.fori_loop(0, CH, fill_ones, 0)
    _zero_vmem_2d(zero_v, BLK, D)
    _blocks(s, lambda off: pltpu.sync_copy(zero_v, acc_sh.at[pl.ds(off, BLK)]))
    pltpu.sync_copy(dst_hbm.at[wid], idx_v)
    plsc.subcore_barrier()

    def batch(i, nch):
        ds = [pltpu.async_copy(ones_v, acc_sh.at[idx_v.at[i * KBD + b]],
                               sem, add=True) for b in range(nch)]
        for d in ds:
            d.wait()

    def body(i, carry):
        batch(i, KBD)
        return carry

    lax.fori_loop(0, NBD, body, 0)
    if KTD:
        batch(NBD, KTD)
    plsc.subcore_barrier()
    _blocks(s, lambda off: pltpu.sync_copy(
        acc_sh.at[pl.ds(off, BLK)], out_hbm.at[c, pl.ds(off, BLK)]))


@functools.partial(
    pl.kernel,
    out_type=jax.ShapeDtypeStruct((NC, N_NODES, D), jnp.float32),
    mesh=_mesh,
    scratch_types=[
        pltpu.VMEM((2, KI, CH), jnp.int32),
        pltpu.VMEM((2, KI, CH), jnp.int32),
        pltpu.VMEM((KR, CH, D), jnp.float32),
        pltpu.VMEM_SHARED((ACC_R, D), jnp.float32),
        pltpu.SemaphoreType.DMA,
        pltpu.SemaphoreType.DMA,
        pltpu.SemaphoreType.DMA,
    ],
)
def _sc_scatter(g_hbm, src_hbm, dst_hbm, out_hbm,
                isrc_v, idst_v, rows_v, acc_sh, gsem, ssem, isem):
    c = lax.axis_index("c")
    s = lax.axis_index("s")
    wid = c * NS + s

    pltpu.async_copy(src_hbm.at[wid, 0], isrc_v.at[0], isem)
    pltpu.async_copy(dst_hbm.at[wid, 0], idst_v.at[0], isem)
    _zero_vmem_2d(rows_v.at[0], BLK, D)
    _blocks(s, lambda off: pltpu.sync_copy(rows_v.at[0, pl.ds(0, BLK)],
                                           acc_sh.at[pl.ds(off, BLK)]))
    plsc.subcore_barrier()

    def idx_wait(p):
        pltpu.make_async_copy(src_hbm.at[0, 0], isrc_v.at[p], isem).wait()
        pltpu.make_async_copy(dst_hbm.at[0, 0], idst_v.at[p], isem).wait()

    def body(m, carry):
        p = lax.rem(m, 2)
        idx_wait(p)
        pltpu.async_copy(src_hbm.at[wid, m + 1], isrc_v.at[1 - p], isem)
        pltpu.async_copy(dst_hbm.at[wid, m + 1], idst_v.at[1 - p], isem)

        def gath(ch, buf):
            return [pltpu.async_copy(g_hbm.at[isrc_v.at[p, ch + i]],
                                     rows_v.at[buf + i], gsem)
                    for i in range(2)]

        def scat(ch, buf):
            return [pltpu.async_copy(rows_v.at[buf + i],
                                     acc_sh.at[idst_v.at[p, ch + i]],
                                     ssem, add=True) for i in range(2)]

        g01 = gath(0, 0)
        for d in g01:
            d.wait()
        s01 = scat(0, 0)
        g23 = gath(2, 2)
        for d in g23:
            d.wait()
        s23 = scat(2, 2)
        for d in s01:
            d.wait()
        g45 = gath(4, 0)
        for d in g45:
            d.wait()
        s45 = scat(4, 0)
        for d in s23:
            d.wait()
        g67 = gath(6, 2)
        for d in g67:
            d.wait()
        s67 = scat(6, 2)
        for d in s45:
            d.wait()
        for d in s67:
            d.wait()
        return carry

    lax.fori_loop(0, NMF, body, 0)

    tp = NMF % 2
    idx_wait(tp)

    def gath1(ch, buf):
        return [pltpu.async_copy(g_hbm.at[isrc_v.at[tp, ch + i]],
                                 rows_v.at[buf + i], gsem) for i in range(2)]

    def scat1(ch, buf):
        return [pltpu.async_copy(rows_v.at[buf + i],
                                 acc_sh.at[idst_v.at[tp, ch + i]],
                                 ssem, add=True) for i in range(2)]

    tg01 = gath1(0, 0)
    for d in tg01:
        d.wait()
    ts01 = scat1(0, 0)
    tg23 = gath1(2, 2)
    for d in tg23:
        d.wait()
    ts23 = scat1(2, 2)
    for d in ts01:
        d.wait()
    tg4 = pltpu.async_copy(g_hbm.at[isrc_v.at[tp, 4]], rows_v.at[0], gsem)
    tg4.wait()
    ts4 = pltpu.async_copy(rows_v.at[0], acc_sh.at[idst_v.at[tp, 4]],
                           ssem, add=True)
    for d in ts23:
        d.wait()
    ts4.wait()
    plsc.subcore_barrier()
    _blocks(s, lambda off: pltpu.sync_copy(
        acc_sh.at[pl.ds(off, BLK)], out_hbm.at[c, pl.ds(off, BLK)]))


_BR = 1000
_GRID = N_NODES // _BR


def _dis_of(degp):
    return lax.rsqrt(degp[0] + degp[1] + 1.0)


def _tc1_body(x_ref, w_ref, degp_ref, o_ref):
    dis = _dis_of(degp_ref[...])
    h = jnp.dot(x_ref[...], w_ref[...], preferred_element_type=jnp.float32)
    o_ref[...] = h * dis


def _tc2_body(s_ref, g_ref, degp_ref, b_ref, w_ref, o_ref):
    dis = _dis_of(degp_ref[...])
    tot = s_ref[0] + s_ref[1] + g_ref[...]
    z = jnp.maximum(tot * dis + b_ref[...], 0.0)
    h = jnp.dot(z, w_ref[...], preferred_element_type=jnp.float32)
    o_ref[...] = h * dis


def _tc3_body(s_ref, g_ref, degp_ref, b_ref, o_ref):
    dis = _dis_of(degp_ref[...])
    tot = s_ref[0] + s_ref[1] + g_ref[...]
    o_ref[...] = jnp.maximum(tot * dis + b_ref[...], 0.0)


_spec_rows = pl.BlockSpec((_BR, D), lambda i: (i, 0))
_spec_S = pl.BlockSpec((NC, _BR, D), lambda i: (0, i, 0))
_spec_degp = pl.BlockSpec((NC, _BR, D), lambda i: (0, i, 0))
_spec_w = pl.BlockSpec((D, D), lambda i: (0, 0))
_spec_b = pl.BlockSpec((1, D), lambda i: (0, 0))

_tc1 = pl.pallas_call(
    _tc1_body, grid=(_GRID,),
    in_specs=[_spec_rows, _spec_w, _spec_degp],
    out_specs=_spec_rows,
    out_shape=jax.ShapeDtypeStruct((N_NODES, D), jnp.float32),
)
_tc2 = pl.pallas_call(
    _tc2_body, grid=(_GRID,),
    in_specs=[_spec_S, _spec_rows, _spec_degp, _spec_b, _spec_w],
    out_specs=_spec_rows,
    out_shape=jax.ShapeDtypeStruct((N_NODES, D), jnp.float32),
)
_tc3 = pl.pallas_call(
    _tc3_body, grid=(_GRID,),
    in_specs=[_spec_S, _spec_rows, _spec_degp, _spec_b],
    out_specs=_spec_rows,
    out_shape=jax.ShapeDtypeStruct((N_NODES, D), jnp.float32),
)


@jax.jit
def kernel(x, edge_index, W1, b1, W2, b2):
    pad = ((0, 0), (0, EPTP - EPT))
    src_p = jnp.pad(edge_index[0].astype(jnp.int32).reshape(NW, EPT), pad)
    dst_p = jnp.pad(edge_index[1].astype(jnp.int32).reshape(NW, EPT), pad)
    src4 = src_p.reshape(NW, NM, KI, CH)
    dst4 = dst_p.reshape(NW, NM, KI, CH)
    dst3 = dst_p.reshape(NW, NCHP, CH)
    b1r = b1.reshape(1, D)
    b2r = b2.reshape(1, D)

    degp = _sc_degree(dst3)
    g1 = _tc1(x, W1, degp)
    S1 = _sc_scatter(g1, src4, dst4)
    g2 = _tc2(S1, g1, degp, b1r, W2)
    S2 = _sc_scatter(g2, src4, dst4)
    return _tc3(S2, g2, degp, b2r)


# speedup vs baseline: 3.1769x; 1.0564x over previous
import functools
import jax
import jax.numpy as jnp
from jax import lax
from jax.experimental import pallas as pl
from jax.experimental.pallas import tpu as pltpu
from jax.experimental.pallas import tpu_sc as plsc

N_NODES = 10000
N_EDGES = 320000
D = 128
NC, NS = 2, 16
NW = NC * NS
CH = 80
EPT = N_EDGES // NW
NCH = EPT // CH
KI = 8
NM = -(-NCH // KI)
KR = 4
NMF = NCH // KI
KTL = NCH - NMF * KI
EPTP = NM * KI * CH
NCHP = NM * KI
ACC_R = N_NODES
KBD = 10
NBD = NCH // KBD
KTD = NCH - NBD * KBD
BLK = 40
NBLK = N_NODES // BLK
BPT = -(-NBLK // NS)

_mesh = plsc.VectorSubcoreMesh(core_axis_name="c", subcore_axis_name="s")


def _zero_vmem_2d(ref, nrows, ncols):
    z16 = jnp.zeros((16,), jnp.float32)

    def row(r, carry):
        for j in range(ncols // 16):
            ref[r, pl.ds(j * 16, 16)] = z16
        return carry

    lax.fori_loop(0, nrows, row, 0)


def _blocks(s, fn):
    def body(j, carry):
        bi = s + j * NS

        @pl.when(bi < NBLK)
        def _():
            fn(pl.multiple_of(bi * BLK, 8))

        return carry

    lax.fori_loop(0, BPT, body, 0)


def _drain_spair(acc_sh, rows_v, ssem):
    for _ in range(2):
        pltpu.make_async_copy(rows_v.at[2], acc_sh.at[pl.ds(0, CH)],
                              ssem).wait()


@functools.partial(
    pl.kernel,
    out_type=jax.ShapeDtypeStruct((NC, N_NODES, D), jnp.float32),
    mesh=_mesh,
    scratch_types=[
        pltpu.VMEM((NCHP, CH), jnp.int32),
        pltpu.VMEM((CH, D), jnp.float32),
        pltpu.VMEM((BLK, D), jnp.float32),
        pltpu.VMEM_SHARED((ACC_R, D), jnp.float32),
        pltpu.SemaphoreType.DMA,
    ],
)
def _sc_degree(dst_hbm, out_hbm, idx_v, ones_v, zero_v, acc_sh, sem):
    c = lax.axis_index("c")
    s = lax.axis_index("s")
    wid = c * NS + s

    one16 = jnp.ones((16,), jnp.float32)

    def fill_ones(r, carry):
        for j in range(D // 16):
            ones_v[r, pl.ds(j * 16, 16)] = one16
        return carry

    lax.fori_loop(0, CH, fill_ones, 0)
    _zero_vmem_2d(zero_v, BLK, D)
    _blocks(s, lambda off: pltpu.sync_copy(zero_v, acc_sh.at[pl.ds(off, BLK)]))
    pltpu.sync_copy(dst_hbm.at[wid], idx_v)
    plsc.subcore_barrier()

    def batch(i, nch):
        ds = [pltpu.async_copy(ones_v, acc_sh.at[idx_v.at[i * KBD + b]],
                               sem, add=True) for b in range(nch)]
        for d in ds:
            d.wait()

    def body(i, carry):
        batch(i, KBD)
        return carry

    lax.fori_loop(0, NBD, body, 0)
    if KTD:
        batch(NBD, KTD)
    plsc.subcore_barrier()
    _blocks(s, lambda off: pltpu.sync_copy(
        acc_sh.at[pl.ds(off, BLK)], out_hbm.at[c, pl.ds(off, BLK)]))


@functools.partial(
    pl.kernel,
    out_type=jax.ShapeDtypeStruct((NC, N_NODES, D), jnp.float32),
    mesh=_mesh,
    scratch_types=[
        pltpu.VMEM((2, KI, CH), jnp.int32),
        pltpu.VMEM((2, KI, CH), jnp.int32),
        pltpu.VMEM((KR, CH, D), jnp.float32),
        pltpu.VMEM_SHARED((ACC_R, D), jnp.float32),
        pltpu.SemaphoreType.DMA,
        pltpu.SemaphoreType.DMA,
        pltpu.SemaphoreType.DMA,
    ],
)
def _sc_scatter(g_hbm, src_hbm, dst_hbm, out_hbm,
                isrc_v, idst_v, rows_v, acc_sh, gsem, ssem, isem):
    c = lax.axis_index("c")
    s = lax.axis_index("s")
    wid = c * NS + s

    pltpu.async_copy(src_hbm.at[wid, 0], isrc_v.at[0], isem)
    pltpu.async_copy(dst_hbm.at[wid, 0], idst_v.at[0], isem)
    _zero_vmem_2d(rows_v.at[0], BLK, D)
    _blocks(s, lambda off: pltpu.sync_copy(rows_v.at[0, pl.ds(0, BLK)],
                                           acc_sh.at[pl.ds(off, BLK)]))
    plsc.subcore_barrier()

    def idx_wait(p):
        pltpu.make_async_copy(src_hbm.at[0, 0], isrc_v.at[p], isem).wait()
        pltpu.make_async_copy(dst_hbm.at[0, 0], idst_v.at[p], isem).wait()

    def body(m, carry):
        p = lax.rem(m, 2)
        idx_wait(p)
        pltpu.async_copy(src_hbm.at[wid, m + 1], isrc_v.at[1 - p], isem)
        pltpu.async_copy(dst_hbm.at[wid, m + 1], idst_v.at[1 - p], isem)

        def gath(ch, buf):
            return [pltpu.async_copy(g_hbm.at[isrc_v.at[p, ch + i]],
                                     rows_v.at[buf + i], gsem)
                    for i in range(2)]

        def scat(ch, buf):
            return [pltpu.async_copy(rows_v.at[buf + i],
                                     acc_sh.at[idst_v.at[p, ch + i]],
                                     ssem, add=True) for i in range(2)]

        g01 = gath(0, 0)
        for d in g01:
            d.wait()
        s01 = scat(0, 0)

        @pl.when(m > 0)
        def _():
            _drain_spair(acc_sh, rows_v, ssem)

        g23 = gath(2, 2)
        for d in g23:
            d.wait()
        s23 = scat(2, 2)
        for d in s01:
            d.wait()
        g45 = gath(4, 0)
        for d in g45:
            d.wait()
        s45 = scat(4, 0)
        for d in s23:
            d.wait()
        g67 = gath(6, 2)
        for d in g67:
            d.wait()
        scat(6, 2)
        for d in s45:
            d.wait()
        return carry

    lax.fori_loop(0, NMF, body, 0)
    _drain_spair(acc_sh, rows_v, ssem)

    tp = NMF % 2
    idx_wait(tp)

    def gath1(ch, buf):
        return [pltpu.async_copy(g_hbm.at[isrc_v.at[tp, ch + i]],
                                 rows_v.at[buf + i], gsem) for i in range(2)]

    def scat1(ch, buf):
        return [pltpu.async_copy(rows_v.at[buf + i],
                                 acc_sh.at[idst_v.at[tp, ch + i]],
                                 ssem, add=True) for i in range(2)]

    tg01 = gath1(0, 0)
    for d in tg01:
        d.wait()
    ts01 = scat1(0, 0)
    tg23 = gath1(2, 2)
    for d in tg23:
        d.wait()
    ts23 = scat1(2, 2)
    for d in ts01:
        d.wait()
    tg4 = pltpu.async_copy(g_hbm.at[isrc_v.at[tp, 4]], rows_v.at[0], gsem)
    tg4.wait()
    ts4 = pltpu.async_copy(rows_v.at[0], acc_sh.at[idst_v.at[tp, 4]],
                           ssem, add=True)
    for d in ts23:
        d.wait()
    ts4.wait()
    plsc.subcore_barrier()
    _blocks(s, lambda off: pltpu.sync_copy(
        acc_sh.at[pl.ds(off, BLK)], out_hbm.at[c, pl.ds(off, BLK)]))


_BR = 1000
_GRID = N_NODES // _BR


def _dis_of(degp):
    return lax.rsqrt(degp[0] + degp[1] + 1.0)


def _tc1_body(x_ref, w_ref, degp_ref, o_ref):
    dis = _dis_of(degp_ref[...])
    h = jnp.dot(x_ref[...], w_ref[...], preferred_element_type=jnp.float32)
    o_ref[...] = h * dis


def _tc2_body(s_ref, g_ref, degp_ref, b_ref, w_ref, o_ref):
    dis = _dis_of(degp_ref[...])
    tot = s_ref[0] + s_ref[1] + g_ref[...]
    z = jnp.maximum(tot * dis + b_ref[...], 0.0)
    h = jnp.dot(z, w_ref[...], preferred_element_type=jnp.float32)
    o_ref[...] = h * dis


def _tc3_body(s_ref, g_ref, degp_ref, b_ref, o_ref):
    dis = _dis_of(degp_ref[...])
    tot = s_ref[0] + s_ref[1] + g_ref[...]
    o_ref[...] = jnp.maximum(tot * dis + b_ref[...], 0.0)


_spec_rows = pl.BlockSpec((_BR, D), lambda i: (i, 0))
_spec_S = pl.BlockSpec((NC, _BR, D), lambda i: (0, i, 0))
_spec_degp = pl.BlockSpec((NC, _BR, D), lambda i: (0, i, 0))
_spec_w = pl.BlockSpec((D, D), lambda i: (0, 0))
_spec_b = pl.BlockSpec((1, D), lambda i: (0, 0))

_tc1 = pl.pallas_call(
    _tc1_body, grid=(_GRID,),
    in_specs=[_spec_rows, _spec_w, _spec_degp],
    out_specs=_spec_rows,
    out_shape=jax.ShapeDtypeStruct((N_NODES, D), jnp.float32),
)
_tc2 = pl.pallas_call(
    _tc2_body, grid=(_GRID,),
    in_specs=[_spec_S, _spec_rows, _spec_degp, _spec_b, _spec_w],
    out_specs=_spec_rows,
    out_shape=jax.ShapeDtypeStruct((N_NODES, D), jnp.float32),
)
_tc3 = pl.pallas_call(
    _tc3_body, grid=(_GRID,),
    in_specs=[_spec_S, _spec_rows, _spec_degp, _spec_b],
    out_specs=_spec_rows,
    out_shape=jax.ShapeDtypeStruct((N_NODES, D), jnp.float32),
)


@jax.jit
def kernel(x, edge_index, W1, b1, W2, b2):
    pad = ((0, 0), (0, EPTP - EPT))
    src_p = jnp.pad(edge_index[0].astype(jnp.int32).reshape(NW, EPT), pad)
    dst_p = jnp.pad(edge_index[1].astype(jnp.int32).reshape(NW, EPT), pad)
    src4 = src_p.reshape(NW, NM, KI, CH)
    dst4 = dst_p.reshape(NW, NM, KI, CH)
    dst3 = dst_p.reshape(NW, NCHP, CH)
    b1r = b1.reshape(1, D)
    b2r = b2.reshape(1, D)

    degp = _sc_degree(dst3)
    g1 = _tc1(x, W1, degp)
    S1 = _sc_scatter(g1, src4, dst4)
    g2 = _tc2(S1, g1, degp, b1r, W2)
    S2 = _sc_scatter(g2, src4, dst4)
    return _tc3(S2, g2, degp, b2r)
